# Initial kernel scaffold; baseline (speedup 1.0000x reference)
import jax
import jax.numpy as jnp
from jax import lax
from jax.experimental import pallas as pl
from jax.experimental.pallas import tpu as pltpu
from jax.experimental.pallas import tpu_sc as plsc

NUM_FIELDS = 26
FIELD_SIZE = 40000
BATCH = 16384
L = 16
NC = 2
NS = 16
NW = NC * NS
B_PER_W = BATCH // NW


def _worker_id():
    return lax.axis_index("s") * NC + lax.axis_index("c")


def _gather_body(xt_hbm, table_hbm, partials_hbm, idx_v, tab_v, out_v):
    wid = _worker_id()

    @pl.when(wid < NUM_FIELDS)
    def _():
        f = wid
        base = pl.multiple_of(f * FIELD_SIZE, 8)
        pltpu.sync_copy(table_hbm.at[pl.ds(base, FIELD_SIZE)], tab_v)
        pltpu.sync_copy(xt_hbm.at[f], idx_v)

        def body(i, carry):
            s = pl.ds(i * L, L)
            out_v[s] = plsc.load_gather(tab_v, [idx_v[s]])
            return carry

        lax.fori_loop(0, BATCH // L, body, 0)
        pltpu.sync_copy(out_v, partials_hbm.at[f])


def _reduce_body(partials_hbm, bias_hbm, out_hbm, cols_v, bias_v, out_v):
    wid = _worker_id()
    base = pl.multiple_of(wid * B_PER_W, 8)
    pltpu.sync_copy(partials_hbm.at[:, pl.ds(base, B_PER_W)], cols_v)
    pltpu.sync_copy(bias_hbm, bias_v)

    def body(c, carry):
        acc = bias_v[:]
        for f in range(NUM_FIELDS):
            acc = acc + cols_v[f, pl.ds(c * L, L)]
        out_v[pl.ds(c * L, L)] = acc
        return carry

    lax.fori_loop(0, B_PER_W // L, body, 0)
    pltpu.sync_copy(out_v, out_hbm.at[pl.ds(base, B_PER_W)])


def _sc_mesh():
    return plsc.VectorSubcoreMesh(core_axis_name="c", subcore_axis_name="s")


@jax.jit
def _run(xt, table, bias16):
    gather = pl.kernel(
        _gather_body,
        out_type=jax.ShapeDtypeStruct((NUM_FIELDS, BATCH), jnp.float32),
        mesh=_sc_mesh(),
        scratch_types=[
            pltpu.VMEM((BATCH,), jnp.int32),
            pltpu.VMEM((FIELD_SIZE,), jnp.float32),
            pltpu.VMEM((BATCH,), jnp.float32),
        ],
        name="features_linear_gather",
    )
    partials = gather(xt, table)

    reduce_k = pl.kernel(
        _reduce_body,
        out_type=jax.ShapeDtypeStruct((BATCH,), jnp.float32),
        mesh=_sc_mesh(),
        scratch_types=[
            pltpu.VMEM((NUM_FIELDS, B_PER_W), jnp.float32),
            pltpu.VMEM((L,), jnp.float32),
            pltpu.VMEM((B_PER_W,), jnp.float32),
        ],
        name="features_linear_reduce",
    )
    return reduce_k(partials, bias16)


def kernel(x, fc_weight, bias):
    xt = x.T.astype(jnp.int32)
    table = fc_weight.reshape(-1)
    bias16 = jnp.broadcast_to(bias.astype(jnp.float32), (L,))
    out = _run(xt, table, bias16)
    return out.reshape(BATCH, 1)


# speedup vs baseline: 1.6079x; 1.6079x over previous
import jax
import jax.numpy as jnp
from jax import lax
from jax.experimental import pallas as pl
from jax.experimental.pallas import tpu as pltpu
from jax.experimental.pallas import tpu_sc as plsc

NUM_FIELDS = 26
FIELD_SIZE = 40000
BATCH = 16384
L = 16
NC = 2
NS = 16
NW = NC * NS
B_PER_W = BATCH // NW


def _worker_id():
    return lax.axis_index("s") * NC + lax.axis_index("c")


def _gather_body(xt_hbm, table_hbm, partials_hbm, idx_v, tab_v, out_v):
    wid = _worker_id()

    @pl.when(wid < NUM_FIELDS)
    def _():
        f = wid
        base = pl.multiple_of(f * FIELD_SIZE, 8)
        pltpu.sync_copy(table_hbm.at[pl.ds(base, FIELD_SIZE)], tab_v)
        pltpu.sync_copy(xt_hbm.at[f], idx_v)

        def body(i, carry):
            s = pl.ds(i * L, L)
            out_v[s] = plsc.load_gather(tab_v, [idx_v[s]])
            return carry

        lax.fori_loop(0, BATCH // L, body, 0)
        pltpu.sync_copy(out_v, partials_hbm.at[f])


def _reduce_body(partials_hbm, bias_hbm, out_hbm, cols_v, bias_v, out_v):
    wid = _worker_id()
    base = pl.multiple_of(wid * B_PER_W, 8)
    pltpu.sync_copy(partials_hbm.at[:, pl.ds(base, B_PER_W)], cols_v)
    pltpu.sync_copy(bias_hbm, bias_v)

    def body(c, carry):
        acc = bias_v[:]
        for f in range(NUM_FIELDS):
            acc = acc + cols_v[f, pl.ds(c * L, L)]
        out_v[pl.ds(c * L, L)] = acc
        return carry

    lax.fori_loop(0, B_PER_W // L, body, 0)
    pltpu.sync_copy(out_v, out_hbm.at[pl.ds(base, B_PER_W)])


def _sc_mesh():
    return plsc.VectorSubcoreMesh(core_axis_name="c", subcore_axis_name="s")


_SC_PARAMS = pltpu.CompilerParams(needs_layout_passes=False)


@jax.jit
def _run(xt, table, bias16):
    gather = pl.kernel(
        _gather_body,
        out_type=jax.ShapeDtypeStruct((NUM_FIELDS, BATCH), jnp.float32),
        mesh=_sc_mesh(),
        scratch_types=[
            pltpu.VMEM((BATCH,), jnp.int32),
            pltpu.VMEM((FIELD_SIZE,), jnp.float32),
            pltpu.VMEM((BATCH,), jnp.float32),
        ],
        name="features_linear_gather",
        compiler_params=_SC_PARAMS,
    )
    partials = gather(xt, table)

    reduce_k = pl.kernel(
        _reduce_body,
        out_type=jax.ShapeDtypeStruct((BATCH,), jnp.float32),
        mesh=_sc_mesh(),
        scratch_types=[
            pltpu.VMEM((NUM_FIELDS, B_PER_W), jnp.float32),
            pltpu.VMEM((L,), jnp.float32),
            pltpu.VMEM((B_PER_W,), jnp.float32),
        ],
        name="features_linear_reduce",
        compiler_params=_SC_PARAMS,
    )
    return reduce_k(partials, bias16)


def kernel(x, fc_weight, bias):
    xt = x.T.astype(jnp.int32)
    table = fc_weight.reshape(-1)
    bias16 = jnp.broadcast_to(bias.astype(jnp.float32), (L,))
    out = _run(xt, table, bias16)
    return out.reshape(BATCH, 1)
